# Initial kernel scaffold; baseline (speedup 1.0000x reference)
import jax
import jax.numpy as jnp
from jax.experimental import pallas as pl


def kernel(edge_rep, distance_matrix, W, b, passenger_idx, driver_idx):
    raise NotImplementedError("write your pallas kernel here")


# speedup vs baseline: 2.1245x; 2.1245x over previous
import functools

import jax
import jax.numpy as jnp
from jax import lax
from jax.experimental import pallas as pl
from jax.experimental.pallas import tpu as pltpu
from jax.experimental.pallas import tpu_sc as plsc

B, L, D, P = 64, 5941, 32, 64
NW = 32
N_REAL = B * P + B
N_PAD = 4352
B_PER_W = N_PAD // NW
CHUNKS = ((0, 128), (128, 8))


_BM = 256


def _dw_body(d_ref, w_ref, b_ref, o_ref):
    o_ref[...] = (
        jnp.dot(
            d_ref[...].astype(jnp.bfloat16),
            w_ref[...].astype(jnp.bfloat16),
            preferred_element_type=jnp.float32,
        )
        + b_ref[...]
    )


def _compute_dw(distance_matrix, W, b):
    grid = (pl.cdiv(L, _BM),)
    return pl.pallas_call(
        _dw_body,
        grid=grid,
        in_specs=[
            pl.BlockSpec((_BM, L), lambda i: (i, 0)),
            pl.BlockSpec((L, D), lambda i: (0, 0)),
            pl.BlockSpec((1, D), lambda i: (0, 0)),
        ],
        out_specs=pl.BlockSpec((_BM, D), lambda i: (i, 0)),
        out_shape=jax.ShapeDtypeStruct((L, D), jnp.float32),
    )(distance_matrix, W, b.reshape(1, D))


def _sc_gather_body(edge_hbm, dw_hbm, gidx_hbm, lidx_hbm, out_e_hbm,
                    out_w_hbm, idx_v, rows_v, sem):
    wid = lax.axis_index("s") * 2 + lax.axis_index("c")
    base = wid * B_PER_W
    for idx_hbm, tab_hbm, out_hbm in (
        (gidx_hbm, edge_hbm, out_e_hbm),
        (lidx_hbm, dw_hbm, out_w_hbm),
    ):
        pltpu.sync_copy(idx_hbm.at[pl.ds(base, B_PER_W)], idx_v)
        for off, cs in CHUNKS:
            pltpu.async_copy(
                tab_hbm.at[idx_v.at[pl.ds(off, cs)]],
                rows_v.at[pl.ds(off, cs)],
                sem,
            ).wait()
        pltpu.sync_copy(rows_v, out_hbm.at[pl.ds(base, B_PER_W)])


def _sc_gather(edge_flat, dw, gidx, lidx):
    mesh = plsc.VectorSubcoreMesh(core_axis_name="c", subcore_axis_name="s")
    kern = functools.partial(
        pl.kernel,
        mesh=mesh,
        out_type=(
            jax.ShapeDtypeStruct((N_PAD, D), jnp.float32),
            jax.ShapeDtypeStruct((N_PAD, D), jnp.float32),
        ),
        scratch_types=[
            pltpu.VMEM((B_PER_W,), jnp.int32),
            pltpu.VMEM((B_PER_W, D), jnp.float32),
            pltpu.SemaphoreType.DMA,
        ],
        compiler_params=pltpu.CompilerParams(use_tc_tiling_on_sc=False),
    )(_sc_gather_body)
    return kern(edge_flat, dw, gidx, lidx)


def _loop_body(pe_ref, pw_ref, de_ref, dw_ref, out_ref, prb_ref, a_ref):
    prb_ref[:, 0:D] = (pe_ref[...] * 0.05).astype(jnp.bfloat16)
    prb_ref[:, D : 2 * D] = (pw_ref[...] * 0.05).astype(jnp.bfloat16)

    def gram(bi, _):
        blk = prb_ref[pl.ds(bi * P, P), :]
        a_ref[pl.ds(bi * P, P), :] = lax.dot_general(
            blk, blk, (((1,), (1,)), ((), ())),
            preferred_element_type=jnp.float32,
        )
        return 0

    lax.fori_loop(0, B, gram, 0)

    drv = jnp.concatenate(
        [(de_ref[...] * 0.05).astype(jnp.bfloat16),
         (dw_ref[...] * 0.05).astype(jnp.bfloat16)], axis=1
    ).astype(jnp.float32)
    pr3 = prb_ref[...].astype(jnp.float32).reshape(B, P, 2 * D)
    attn0 = jnp.sum(pr3 * drv[:, None, :], axis=2)

    iota_p = lax.broadcasted_iota(jnp.int32, (B, P), 1)

    def step(t, carry):
        attn, mask = carry
        mx = jnp.max(attn, axis=1, keepdims=True)
        e = jnp.exp(attn - mx)
        out_ref[pl.ds(t, 1)] = (e / jnp.sum(e, axis=1, keepdims=True))[None]
        masked = attn * mask
        mmax = jnp.max(masked, axis=1, keepdims=True)
        cand = jnp.where(masked == mmax, iota_p, P)
        sel = jnp.min(cand, axis=1, keepdims=True)
        onehot = (iota_p == sel).astype(jnp.float32)
        mask = mask * (1.0 - onehot)
        a3 = a_ref[...].reshape(B, P, P)
        attn = jnp.sum(a3 * onehot[:, :, None], axis=1)
        return attn, mask

    lax.fori_loop(0, P, step, (attn0, jnp.ones((B, P), jnp.float32)))


def _pointer_loop(pe, pw, de, dw):
    return pl.pallas_call(
        _loop_body,
        out_shape=jax.ShapeDtypeStruct((P, B, P), jnp.float32),
        scratch_shapes=[
            pltpu.VMEM((B * P, 2 * D), jnp.bfloat16),
            pltpu.VMEM((B * P, P), jnp.float32),
        ],
    )(pe, pw, de, dw)


def kernel(edge_rep, distance_matrix, W, b, passenger_idx, driver_idx):
    dwt = _compute_dw(distance_matrix, W, b)

    p_flat = passenger_idx.reshape(-1).astype(jnp.int32)
    d_flat = driver_idx.astype(jnp.int32)
    pad = jnp.zeros((N_PAD - N_REAL,), jnp.int32)
    lidx = jnp.concatenate([p_flat, d_flat, pad])
    case_off = (jnp.arange(B, dtype=jnp.int32) * L)
    goff = jnp.concatenate([jnp.repeat(case_off, P), case_off, pad])
    gidx = lidx + goff

    edge_flat = edge_rep.reshape(B * L, D)
    ge, gw = _sc_gather(edge_flat, dwt, gidx, lidx)

    out = _pointer_loop(ge[: B * P], gw[: B * P],
                        ge[B * P : N_REAL], gw[B * P : N_REAL])
    return jnp.transpose(out, (1, 0, 2))


# speedup vs baseline: 2.6726x; 1.2580x over previous
import functools

import jax
import jax.numpy as jnp
from jax import lax
from jax.experimental import pallas as pl
from jax.experimental.pallas import tpu as pltpu
from jax.experimental.pallas import tpu_sc as plsc

B, L, D, P = 64, 5941, 32, 64
LPAD = 5952
NW = 32
N_REAL = B * P + B
N_PAD = 4352
B_PER_W = N_PAD // NW
CHUNKS = ((0, 128), (128, 8))
EF = B * L // 4
WF = LPAD // 4


_BM = 256


def _dw_body(d_ref, w_ref, b_ref, o_ref):
    o_ref[...] = (
        jnp.dot(
            d_ref[...].astype(jnp.bfloat16),
            w_ref[...].astype(jnp.bfloat16),
            preferred_element_type=jnp.float32,
        )
        + b_ref[...]
    )


def _compute_dw(distance_matrix, W, b):
    grid = (pl.cdiv(LPAD, _BM),)
    return pl.pallas_call(
        _dw_body,
        grid=grid,
        in_specs=[
            pl.BlockSpec((_BM, L), lambda i: (i, 0)),
            pl.BlockSpec((L, D), lambda i: (0, 0)),
            pl.BlockSpec((1, D), lambda i: (0, 0)),
        ],
        out_specs=pl.BlockSpec((_BM, D), lambda i: (i, 0)),
        out_shape=jax.ShapeDtypeStruct((LPAD, D), jnp.float32),
    )(distance_matrix, W, b.reshape(1, D))


def _sc_gather_body(edge_hbm, dw_hbm, gidx_hbm, lidx_hbm, out_e_hbm,
                    out_w_hbm, idx_v, rows_v, sem):
    wid = lax.axis_index("s") * 2 + lax.axis_index("c")
    base = wid * B_PER_W
    for idx_hbm, tab_hbm, out_hbm in (
        (gidx_hbm, edge_hbm, out_e_hbm),
        (lidx_hbm, dw_hbm, out_w_hbm),
    ):
        pltpu.sync_copy(idx_hbm.at[pl.ds(base, B_PER_W)], idx_v)
        for off, cs in CHUNKS:
            pltpu.async_copy(
                tab_hbm.at[idx_v.at[pl.ds(off, cs)]],
                rows_v.at[pl.ds(off, cs)],
                sem,
            ).wait()
        pltpu.sync_copy(rows_v, out_hbm.at[pl.ds(base, B_PER_W)])


def _sc_gather(edge_fat, dw_fat, gidx, lidx):
    mesh = plsc.VectorSubcoreMesh(core_axis_name="c", subcore_axis_name="s")
    kern = functools.partial(
        pl.kernel,
        mesh=mesh,
        out_type=(
            jax.ShapeDtypeStruct((N_PAD, 4 * D), jnp.float32),
            jax.ShapeDtypeStruct((N_PAD, 4 * D), jnp.float32),
        ),
        scratch_types=[
            pltpu.VMEM((B_PER_W,), jnp.int32),
            pltpu.VMEM((B_PER_W, 4 * D), jnp.float32),
            pltpu.SemaphoreType.DMA,
        ],
    )(_sc_gather_body)
    return kern(edge_fat, dw_fat, gidx, lidx)


def _quarter(fat, q):
    out = fat[:, 0:D]
    for g in (1, 2, 3):
        out = jnp.where(q == g, fat[:, g * D : (g + 1) * D], out)
    return out


def _loop_body(pe_ref, pw_ref, de_ref, dw_ref, qp_ref, ql_ref, qde_ref,
               qdw_ref, out_ref, prb_ref, a_ref):
    pe = _quarter(pe_ref[...], qp_ref[...])
    pw = _quarter(pw_ref[...], ql_ref[...])
    prb_ref[:, 0:D] = (pe * 0.05).astype(jnp.bfloat16)
    prb_ref[:, D : 2 * D] = (pw * 0.05).astype(jnp.bfloat16)

    def gram(bi, _):
        blk = prb_ref[pl.ds(bi * P, P), :]
        a_ref[pl.ds(bi * P, P), :] = lax.dot_general(
            blk, blk, (((1,), (1,)), ((), ())),
            preferred_element_type=jnp.float32,
        )
        return 0

    lax.fori_loop(0, B, gram, 0)

    de = _quarter(de_ref[...], qde_ref[...])
    dw = _quarter(dw_ref[...], qdw_ref[...])
    drv = jnp.concatenate(
        [(de * 0.05).astype(jnp.bfloat16),
         (dw * 0.05).astype(jnp.bfloat16)], axis=1
    ).astype(jnp.float32)
    pr3 = prb_ref[...].astype(jnp.float32).reshape(B, P, 2 * D)
    attn0 = jnp.sum(pr3 * drv[:, None, :], axis=2)

    iota_p = lax.broadcasted_iota(jnp.int32, (B, P), 1)

    def step(t, carry):
        attn, mask = carry
        mx = jnp.max(attn, axis=1, keepdims=True)
        e = jnp.exp(attn - mx)
        out_ref[pl.ds(t, 1)] = (e / jnp.sum(e, axis=1, keepdims=True))[None]
        masked = attn * mask
        mmax = jnp.max(masked, axis=1, keepdims=True)
        cand = jnp.where(masked == mmax, iota_p, P)
        sel = jnp.min(cand, axis=1, keepdims=True)
        onehot = (iota_p == sel).astype(jnp.float32)
        mask = mask * (1.0 - onehot)
        a3 = a_ref[...].reshape(B, P, P)
        attn = jnp.sum(a3 * onehot[:, :, None], axis=1)
        return attn, mask

    lax.fori_loop(0, P, step, (attn0, jnp.ones((B, P), jnp.float32)))


def _pointer_loop(pe, pw, de, dw, qp, ql, qde, qdw):
    return pl.pallas_call(
        _loop_body,
        out_shape=jax.ShapeDtypeStruct((P, B, P), jnp.float32),
        scratch_shapes=[
            pltpu.VMEM((B * P, 2 * D), jnp.bfloat16),
            pltpu.VMEM((B * P, P), jnp.float32),
        ],
    )(pe, pw, de, dw, qp, ql, qde, qdw)


def kernel(edge_rep, distance_matrix, W, b, passenger_idx, driver_idx):
    dwt = _compute_dw(distance_matrix, W, b)

    p_flat = passenger_idx.reshape(-1).astype(jnp.int32)
    d_flat = driver_idx.astype(jnp.int32)
    pad = jnp.zeros((N_PAD - N_REAL,), jnp.int32)
    lidx = jnp.concatenate([p_flat, d_flat, pad])
    case_off = (jnp.arange(B, dtype=jnp.int32) * L)
    goff = jnp.concatenate([jnp.repeat(case_off, P), case_off, pad])
    gidx = lidx + goff

    edge_fat = edge_rep.reshape(EF, 4 * D)
    dw_fat = dwt.reshape(WF, 4 * D)
    ge, gw = _sc_gather(edge_fat, dw_fat, gidx // 4, lidx // 4)

    qg = (gidx % 4).reshape(N_PAD, 1)
    ql = (lidx % 4).reshape(N_PAD, 1)
    out = _pointer_loop(
        ge[: B * P], gw[: B * P], ge[B * P : N_REAL], gw[B * P : N_REAL],
        qg[: B * P], ql[: B * P], qg[B * P : N_REAL], ql[B * P : N_REAL],
    )
    return jnp.transpose(out, (1, 0, 2))


# speedup vs baseline: 2.6922x; 1.0073x over previous
import functools

import jax
import jax.numpy as jnp
from jax import lax
from jax.experimental import pallas as pl
from jax.experimental.pallas import tpu as pltpu
from jax.experimental.pallas import tpu_sc as plsc

B, L, D, P = 64, 5941, 32, 64
LPAD = 5952
NW = 32
N_REAL = B * P + B
N_PAD = 4352
B_PER_W = N_PAD // NW
CHUNKS = ((0, 128), (128, 8))
EF = B * L // 4
WF = LPAD // 4


_BM = 256


def _dw_body(d_ref, w_ref, b_ref, o_ref):
    o_ref[...] = (
        jnp.dot(
            d_ref[...].astype(jnp.bfloat16),
            w_ref[...].astype(jnp.bfloat16),
            preferred_element_type=jnp.float32,
        )
        + b_ref[...]
    )


def _compute_dw(distance_matrix, W, b):
    grid = (pl.cdiv(LPAD, _BM),)
    return pl.pallas_call(
        _dw_body,
        grid=grid,
        in_specs=[
            pl.BlockSpec((_BM, L), lambda i: (i, 0)),
            pl.BlockSpec((L, D), lambda i: (0, 0)),
            pl.BlockSpec((1, D), lambda i: (0, 0)),
        ],
        out_specs=pl.BlockSpec((_BM, D), lambda i: (i, 0)),
        out_shape=jax.ShapeDtypeStruct((LPAD, D), jnp.float32),
    )(distance_matrix, W, b.reshape(1, D))


def _sc_gather_body(tab_hbm, idx_hbm, out_hbm, idx_v, rows_v, sem):
    wid = lax.axis_index("s") * 2 + lax.axis_index("c")
    base = wid * B_PER_W
    pltpu.sync_copy(idx_hbm.at[pl.ds(base, B_PER_W)], idx_v)
    for off, cs in CHUNKS:
        pltpu.async_copy(
            tab_hbm.at[idx_v.at[pl.ds(off, cs)]],
            rows_v.at[pl.ds(off, cs)],
            sem,
        ).wait()
    pltpu.sync_copy(rows_v, out_hbm.at[pl.ds(base, B_PER_W)])


def _sc_gather(table_fat, idx):
    mesh = plsc.VectorSubcoreMesh(core_axis_name="c", subcore_axis_name="s")
    kern = functools.partial(
        pl.kernel,
        mesh=mesh,
        out_type=jax.ShapeDtypeStruct((N_PAD, 4 * D), jnp.float32),
        scratch_types=[
            pltpu.VMEM((B_PER_W,), jnp.int32),
            pltpu.VMEM((B_PER_W, 4 * D), jnp.float32),
            pltpu.SemaphoreType.DMA,
        ],
    )(_sc_gather_body)
    return kern(table_fat, idx)


def _quarter(fat, q):
    out = fat[:, 0:D]
    for g in (1, 2, 3):
        out = jnp.where(q == g, fat[:, g * D : (g + 1) * D], out)
    return out


def _loop_body(pe_ref, pw_ref, de_ref, dw_ref, qp_ref, ql_ref, qde_ref,
               qdw_ref, out_ref, prb_ref, a_ref):
    pe = _quarter(pe_ref[...], qp_ref[...])
    pw = _quarter(pw_ref[...], ql_ref[...])
    prb_ref[:, 0:D] = (pe * 0.05).astype(jnp.bfloat16)
    prb_ref[:, D : 2 * D] = (pw * 0.05).astype(jnp.bfloat16)

    def gram(bi, _):
        blk = prb_ref[pl.ds(bi * P, P), :]
        a_ref[pl.ds(bi * P, P), :] = lax.dot_general(
            blk, blk, (((1,), (1,)), ((), ())),
            preferred_element_type=jnp.float32,
        )
        return 0

    lax.fori_loop(0, B, gram, 0)

    de = _quarter(de_ref[...], qde_ref[...])
    dw = _quarter(dw_ref[...], qdw_ref[...])
    drv = jnp.concatenate(
        [(de * 0.05).astype(jnp.bfloat16),
         (dw * 0.05).astype(jnp.bfloat16)], axis=1
    ).astype(jnp.float32)
    pr3 = prb_ref[...].astype(jnp.float32).reshape(B, P, 2 * D)
    attn0 = jnp.sum(pr3 * drv[:, None, :], axis=2)

    iota_p = lax.broadcasted_iota(jnp.int32, (B, P), 1)

    def step(t, carry):
        attn, mask = carry
        mx = jnp.max(attn, axis=1, keepdims=True)
        e = jnp.exp(attn - mx)
        out_ref[pl.ds(t, 1)] = (e / jnp.sum(e, axis=1, keepdims=True))[None]
        masked = attn * mask
        mmax = jnp.max(masked, axis=1, keepdims=True)
        cand = jnp.where(masked == mmax, iota_p, P)
        sel = jnp.min(cand, axis=1, keepdims=True)
        onehot = (iota_p == sel).astype(jnp.float32)
        mask = mask * (1.0 - onehot)
        a3 = a_ref[...].reshape(B, P, P)
        attn = jnp.sum(a3 * onehot[:, :, None], axis=1)
        return attn, mask

    lax.fori_loop(0, P, step, (attn0, jnp.ones((B, P), jnp.float32)))


def _pointer_loop(pe, pw, de, dw, qp, ql, qde, qdw):
    return pl.pallas_call(
        _loop_body,
        out_shape=jax.ShapeDtypeStruct((P, B, P), jnp.float32),
        scratch_shapes=[
            pltpu.VMEM((B * P, 2 * D), jnp.bfloat16),
            pltpu.VMEM((B * P, P), jnp.float32),
        ],
    )(pe, pw, de, dw, qp, ql, qde, qdw)


def kernel(edge_rep, distance_matrix, W, b, passenger_idx, driver_idx):
    p_flat = passenger_idx.reshape(-1).astype(jnp.int32)
    d_flat = driver_idx.astype(jnp.int32)
    pad = jnp.zeros((N_PAD - N_REAL,), jnp.int32)
    lidx = jnp.concatenate([p_flat, d_flat, pad])
    case_off = (jnp.arange(B, dtype=jnp.int32) * L)
    goff = jnp.concatenate([jnp.repeat(case_off, P), case_off, pad])
    gidx = lidx + goff

    edge_fat = edge_rep.reshape(EF, 4 * D)
    ge = _sc_gather(edge_fat, gidx // 4)
    dwt = _compute_dw(distance_matrix, W, b)
    gw = _sc_gather(dwt.reshape(WF, 4 * D), lidx // 4)

    qg = (gidx % 4).reshape(N_PAD, 1)
    ql = (lidx % 4).reshape(N_PAD, 1)
    out = _pointer_loop(
        ge[: B * P], gw[: B * P], ge[B * P : N_REAL], gw[B * P : N_REAL],
        qg[: B * P], ql[: B * P], qg[B * P : N_REAL], ql[B * P : N_REAL],
    )
    return jnp.transpose(out, (1, 0, 2))
